# baseline (speedup 1.0000x reference)
import functools
import math

import jax
import jax.numpy as jnp
from jax import lax
from jax.experimental import pallas as pl
from jax.experimental.pallas import tpu as pltpu
from jax.experimental.pallas import tpu_sc as plsc


def _make_gather(V, D, S, B, scale):
    N = S * B
    info = plsc.get_sparse_core_info()
    NC, NS, L = info.num_cores, info.num_subcores, info.num_lanes
    NW = NC * NS
    CH = 128
    NBUF = 4
    GROUP = CH * NBUF
    UNROLL = 8
    per_w = N // NW
    assert per_w % GROUP == 0 and D % L == 0 and GROUP % UNROLL == 0
    assert B % GROUP == 0
    n_chunks = per_w // CH
    n_groups = per_w // GROUP
    assert n_groups % 2 == 0 and n_groups >= 4

    mesh = plsc.VectorSubcoreMesh(core_axis_name="c", subcore_axis_name="s")

    @functools.partial(
        pl.kernel,
        out_type=jax.ShapeDtypeStruct((S, B, D), jnp.float32),
        mesh=mesh,
        scratch_types=[
            pltpu.VMEM((n_chunks, CH), jnp.int32),
            pltpu.VMEM((GROUP, D), jnp.float32),
            pltpu.VMEM((GROUP, D), jnp.float32),
            pltpu.SemaphoreType.DMA,
            pltpu.SemaphoreType.DMA,
            pltpu.SemaphoreType.DMA,
            pltpu.SemaphoreType.DMA,
        ],
        compiler_params=pltpu.CompilerParams(use_tc_tiling_on_sc=False),
    )
    def gather(table_hbm, idx_hbm, out_hbm, idx_v, buf0, buf1, gs0, gs1, ws0, ws1):
        wid = lax.axis_index("s") * NC + lax.axis_index("c")
        base = wid * per_w
        pltpu.sync_copy(idx_hbm.at[wid], idx_v)

        def fire_gathers(g, buf, gsem):
            for b in range(NBUF):
                j = g * NBUF + b
                pltpu.async_copy(
                    table_hbm.at[idx_v.at[j]], buf.at[pl.ds(b * CH, CH)], gsem
                )

        def drain_gathers(buf, gsem):
            pltpu.make_async_copy(table_hbm.at[pl.ds(0, GROUP)], buf, gsem).wait()

        def scale_bank(buf):
            def sbody(i, carry):
                r0 = i * UNROLL
                for dr in range(UNROLL):
                    for c in range(D // L):
                        sl = pl.ds(c * L, L)
                        buf[r0 + dr, sl] = buf[r0 + dr, sl] * scale
                return carry

            lax.fori_loop(0, GROUP // UNROLL, sbody, 0)

        def fire_write(g, buf, wsem):
            flat = base + g * GROUP
            s_idx = lax.div(flat, B)
            b0 = lax.rem(flat, B)
            pltpu.async_copy(buf, out_hbm.at[s_idx, pl.ds(b0, GROUP)], wsem)

        def drain_write(buf, wsem):
            pltpu.make_async_copy(buf, out_hbm.at[0, pl.ds(0, GROUP)], wsem).wait()

        fire_gathers(0, buf0, gs0)
        fire_gathers(1, buf1, gs1)
        drain_gathers(buf0, gs0)
        scale_bank(buf0)
        fire_write(0, buf0, ws0)

        def body(p, carry):
            g = 1 + 2 * p
            drain_gathers(buf1, gs1)
            scale_bank(buf1)
            fire_write(g, buf1, ws1)
            drain_write(buf0, ws0)
            fire_gathers(g + 1, buf0, gs0)
            g2 = g + 1
            drain_gathers(buf0, gs0)
            scale_bank(buf0)
            fire_write(g2, buf0, ws0)
            drain_write(buf1, ws1)
            fire_gathers(g2 + 1, buf1, gs1)
            return carry

        lax.fori_loop(0, (n_groups - 2) // 2, body, 0)

        drain_gathers(buf1, gs1)
        scale_bank(buf1)
        fire_write(n_groups - 1, buf1, ws1)
        drain_write(buf0, ws0)
        drain_write(buf1, ws1)

    return gather, NW, n_chunks, CH


def kernel(tokens, embedding):
    V, D = embedding.shape
    S, B = tokens.shape
    N = S * B
    scale = float(math.sqrt(D))

    gather, NW, n_chunks, CH = _make_gather(V, D, S, B, scale)
    idx3 = tokens.reshape(NW, n_chunks, CH).astype(jnp.int32)
    return gather(embedding, idx3)


# speedup vs baseline: 3.0573x; 3.0573x over previous
import functools
import math

import jax
import jax.numpy as jnp
from jax import lax
from jax.experimental import pallas as pl
from jax.experimental.pallas import tpu as pltpu
from jax.experimental.pallas import tpu_sc as plsc


def _make_gather(V, D, S, B, scale):
    N = S * B
    info = plsc.get_sparse_core_info()
    NC, NS, L = info.num_cores, info.num_subcores, info.num_lanes
    NW = NC * NS
    CH = 128
    NBUF = 2
    GROUP = CH * NBUF
    SUB = 8
    DT = D // SUB
    per_w = N // NW
    assert per_w % GROUP == 0 and D % L == 0
    assert B % GROUP == 0
    n_chunks = per_w // CH
    n_groups = per_w // GROUP
    assert (n_groups - 4) % 3 == 0 and n_groups >= 7

    mesh = plsc.VectorSubcoreMesh(core_axis_name="c", subcore_axis_name="s")

    @functools.partial(
        pl.kernel,
        out_type=jax.ShapeDtypeStruct((S, DT, B // CH, SUB, CH), jnp.float32),
        mesh=mesh,
        scratch_types=[
            pltpu.VMEM((n_chunks, CH), jnp.int32),
            pltpu.VMEM((GROUP, D), jnp.float32),
            pltpu.VMEM((GROUP, D), jnp.float32),
            pltpu.VMEM((GROUP, D), jnp.float32),
            pltpu.VMEM((DT, NBUF, SUB, CH + 1), jnp.float32),
            pltpu.VMEM((DT, NBUF, SUB, CH + 1), jnp.float32),
            pltpu.VMEM((DT, NBUF, SUB, CH + 1), jnp.float32),
            pltpu.SemaphoreType.DMA,
            pltpu.SemaphoreType.DMA,
            pltpu.SemaphoreType.DMA,
            pltpu.SemaphoreType.DMA,
            pltpu.SemaphoreType.DMA,
            pltpu.SemaphoreType.DMA,
        ],
        compiler_params=pltpu.CompilerParams(
            use_tc_tiling_on_sc=False, needs_layout_passes=False
        ),
    )
    def gather(table_hbm, idx_hbm, out_hbm, idx_v, buf0, buf1, buf2,
               st0, st1, st2, gs0, gs1, gs2, ws0, ws1, ws2):
        wid = lax.axis_index("s") * NC + lax.axis_index("c")
        base = wid * per_w
        pltpu.sync_copy(idx_hbm.at[wid], idx_v)
        lanes = lax.iota(jnp.int32, L)

        def fire_gathers(g, buf, gsem):
            for b in range(NBUF):
                j = g * NBUF + b
                pltpu.async_copy(
                    table_hbm.at[idx_v.at[j]], buf.at[pl.ds(b * CH, CH)], gsem
                )

        def drain_gathers(buf, gsem):
            for b in range(NBUF):
                pltpu.make_async_copy(
                    table_hbm.at[pl.ds(0, CH)], buf.at[pl.ds(b * CH, CH)], gsem
                ).wait()

        i_dt = [c * (L // SUB) + lanes // SUB for c in range(D // L)]
        i_dl = lanes % SUB

        def transpose_bank(buf, st):
            @plsc.parallel_loop(0, GROUP, 1, unroll=8)
            def tbody(b):
                bt = b // CH
                bl = b % CH
                v_bt = jnp.broadcast_to(bt, (L,))
                v_bl = jnp.broadcast_to(bl, (L,))
                for c in range(D // L):
                    vals = buf[b, pl.ds(c * L, L)] * scale
                    plsc.store_scatter(st, [i_dt[c], v_bt, i_dl, v_bl], vals)

        def fire_write(g, st, wsem):
            flat = base + g * GROUP
            s_idx = lax.div(flat, B)
            bt0 = lax.div(lax.rem(flat, B), CH)
            pltpu.async_copy(
                st.at[:, :, :, pl.ds(0, CH)],
                out_hbm.at[s_idx, :, pl.ds(bt0, NBUF)],
                wsem,
            )

        def drain_write(st, wsem):
            pltpu.make_async_copy(
                st.at[:, :, :, pl.ds(0, CH)],
                out_hbm.at[0, :, pl.ds(0, NBUF)],
                wsem,
            ).wait()

        bufs = (buf0, buf1, buf2)
        sts = (st0, st1, st2)
        gss = (gs0, gs1, gs2)
        wss = (ws0, ws1, ws2)

        def step(g, b, fire_next):
            o = (b + 2) % 3
            if fire_next:
                fire_gathers(g + 2, bufs[o], gss[o])
            drain_gathers(bufs[b], gss[b])
            drain_write(sts[o], wss[o])
            transpose_bank(bufs[b], sts[b])
            fire_write(g, sts[b], wss[b])

        fire_gathers(0, buf0, gs0)
        fire_gathers(1, buf1, gs1)
        fire_gathers(2, buf2, gs2)
        drain_gathers(buf0, gs0)
        transpose_bank(buf0, st0)
        fire_write(0, st0, ws0)

        def body(p, carry):
            g = 1 + 3 * p
            step(g, 1, True)
            step(g + 1, 2, True)
            step(g + 2, 0, True)
            return carry

        lax.fori_loop(0, (n_groups - 4) // 3, body, 0)

        step(n_groups - 3, (n_groups - 3) % 3, True)
        step(n_groups - 2, (n_groups - 2) % 3, False)
        step(n_groups - 1, (n_groups - 1) % 3, False)
        drain_write(sts[(n_groups - 1) % 3], wss[(n_groups - 1) % 3])

    return gather, NW, n_chunks, CH, SUB, DT


def kernel(tokens, embedding):
    V, D = embedding.shape
    S, B = tokens.shape
    scale = float(math.sqrt(D))

    gather, NW, n_chunks, CH, SUB, DT = _make_gather(V, D, S, B, scale)
    idx3 = tokens.reshape(NW, n_chunks, CH).astype(jnp.int32)
    out5 = gather(embedding, idx3)
    return out5.transpose(0, 2, 4, 1, 3).reshape(S, B, D)
